# Initial kernel scaffold; baseline (speedup 1.0000x reference)
import functools

import jax
import jax.numpy as jnp
from jax import lax
from jax.experimental import pallas as pl
from jax.experimental.pallas import tpu as pltpu
from jax.experimental.pallas import tpu_sc as plsc

N = 10000
E = 320000
D = 128
HID = 128
OUT = 2
G = 16
EPS = 1e-5

NC = 2
NS = 16
NW = NC * NS
CHUNK = 128
CHUNKS = 80
EW = CHUNK * CHUNKS
EP = EW * NW
RPT = 626
NP = RPT * NS

_mesh = plsc.VectorSubcoreMesh(core_axis_name="c", subcore_axis_name="s")


@functools.partial(
    pl.kernel,
    mesh=_mesh,
    out_type=jax.ShapeDtypeStruct((NW, NP), jnp.float32),
    scratch_types=[
        pltpu.VMEM((CHUNKS, CHUNK), jnp.int32),
        pltpu.VMEM((NP,), jnp.float32),
    ],
)
def _deg_sc(dst_hbm, out_hbm, dst_v, hist_v):
    c = lax.axis_index("c")
    s = lax.axis_index("s")
    wid = c * NS + s
    pltpu.sync_copy(dst_hbm.at[pl.ds(wid * CHUNKS, CHUNKS)], dst_v)

    zeros16 = jnp.zeros((16,), jnp.float32)

    def zbody(i, carry):
        hist_v[pl.ds(i * 16, 16)] = zeros16
        return carry

    lax.fori_loop(0, NP // 16, zbody, 0)

    ones16 = jnp.ones((16,), jnp.float32)

    def ebody(i, carry):
        row = i // (CHUNK // 16)
        grp = i % (CHUNK // 16)
        idx = dst_v[row, pl.ds(grp * 16, 16)]
        plsc.addupdate_scatter(hist_v, [idx], ones16)
        return carry

    lax.fori_loop(0, CHUNKS * (CHUNK // 16), ebody, 0)
    pltpu.sync_copy(hist_v, out_hbm.at[wid])


@functools.partial(
    pl.kernel,
    mesh=_mesh,
    out_type=jax.ShapeDtypeStruct((NC, NP, HID), jnp.float32),
    scratch_types=[
        pltpu.VMEM((CHUNKS, CHUNK), jnp.int32),
        pltpu.VMEM((CHUNKS, CHUNK), jnp.int32),
        pltpu.VMEM((CHUNK, HID), jnp.float32),
        pltpu.VMEM((CHUNK, HID), jnp.float32),
        pltpu.VMEM_SHARED((NP, HID), jnp.float32),
        pltpu.SemaphoreType.DMA,
        pltpu.SemaphoreType.DMA,
    ],
)
def _agg_sc(y_hbm, src_hbm, dst_hbm, zero_hbm, out_hbm,
            src_v, dst_v, rows0, rows1, acc_sh, sem0, sem1):
    c = lax.axis_index("c")
    s = lax.axis_index("s")
    wid = c * NS + s
    pltpu.sync_copy(zero_hbm, acc_sh.at[pl.ds(s * RPT, RPT)])
    pltpu.sync_copy(src_hbm.at[pl.ds(wid * CHUNKS, CHUNKS)], src_v)
    pltpu.sync_copy(dst_hbm.at[pl.ds(wid * CHUNKS, CHUNKS)], dst_v)
    plsc.subcore_barrier()

    pltpu.async_copy(y_hbm.at[src_v.at[0]], rows0, sem0)

    def body(i, carry):
        for p, buf, sem, obuf, osem in ((0, rows0, sem0, rows1, sem1),
                                        (1, rows1, sem1, rows0, sem0)):
            j = i * 2 + p
            nxt = j + 1

            @pl.when(nxt < CHUNKS)
            def _():
                pltpu.async_copy(y_hbm.at[src_v.at[nxt]], obuf, osem)

            pltpu.make_async_copy(y_hbm.at[src_v.at[j]], buf, sem).wait()
            pltpu.sync_copy(buf, acc_sh.at[dst_v.at[j]], add=True)
        return carry

    lax.fori_loop(0, CHUNKS // 2, body, 0)
    plsc.subcore_barrier()
    pltpu.sync_copy(acc_sh.at[pl.ds(s * RPT, RPT)],
                    out_hbm.at[c, pl.ds(s * RPT, RPT)])


def _dinv_tc(degp_ref, o_ref):
    deg = jnp.sum(degp_ref[...], axis=0, keepdims=True) + 1.0
    lane = lax.broadcasted_iota(jnp.int32, (1, NP), 1)
    o_ref[...] = jnp.where(lane < N, lax.rsqrt(deg), 0.0)


def _prescale_tc(h_ref, w_ref, dinv_ref, o_ref):
    o_ref[...] = (
        jnp.dot(h_ref[...], w_ref[...], preferred_element_type=jnp.float32)
        * dinv_ref[...]
    )


def _bn_relu(z, g_row, beta_row):
    rmask = lax.broadcasted_iota(jnp.int32, (NP, 1), 0) < N
    zm = jnp.where(rmask, z, 0.0)
    mean = jnp.sum(zm, axis=0, keepdims=True) * (1.0 / N)
    cent = z - mean
    var = jnp.sum(jnp.where(rmask, cent * cent, 0.0), axis=0,
                  keepdims=True) * (1.0 / N)
    return jnp.maximum(cent * lax.rsqrt(var + EPS) * g_row + beta_row, 0.0)


def _make_post_tc(with_res):
    def body(*refs):
        if with_res:
            (a_ref, y_ref, dinv_ref, b_ref, g_ref, beta_ref, w_ref,
             res_ref, h_out, y_out) = refs
        else:
            (a_ref, y_ref, dinv_ref, b_ref, g_ref, beta_ref, w_ref,
             h_out, y_out) = refs
        dinv = dinv_ref[...]
        z = dinv * (a_ref[0] + a_ref[1] + y_ref[...]) + b_ref[...]
        h = _bn_relu(z, g_ref[...], beta_ref[...])
        if with_res:
            h = h + res_ref[...]
        h_out[...] = h
        y_out[...] = (
            jnp.dot(h, w_ref[...], preferred_element_type=jnp.float32) * dinv
        )
    return body


def _final_tc(a_ref, y_ref, dinv_ref, b_ref, g_ref, beta_ref, batch_ref,
              wf1_ref, bf1_ref, wf2_ref, bf2_ref, o_ref):
    z = dinv_ref[...] * (a_ref[0] + a_ref[1] + y_ref[...]) + b_ref[...]
    h = _bn_relu(z, g_ref[...], beta_ref[...])
    hn = h[:N]
    bc = batch_ref[...]
    gid = lax.broadcasted_iota(jnp.int32, (1, G), 1)
    onehot = (bc == gid).astype(jnp.float32)
    seg_sum = lax.dot_general(onehot, hn, (((0,), (0,)), ((), ())),
                              preferred_element_type=jnp.float32)
    cnt = lax.dot_general(onehot, jnp.ones((N, 1), jnp.float32),
                          (((0,), (0,)), ((), ())),
                          preferred_element_type=jnp.float32)
    mean_pool = seg_sum / jnp.maximum(cnt, 1.0)
    max_pool = jnp.concatenate(
        [jnp.max(jnp.where(bc == gi, hn, -jnp.inf), axis=0, keepdims=True)
         for gi in range(G)], axis=0)
    pooled = jnp.concatenate([mean_pool, max_pool], axis=1)
    hfc = jnp.maximum(
        jnp.dot(pooled, wf1_ref[...], preferred_element_type=jnp.float32)
        + bf1_ref[...], 0.0)
    o_ref[...] = (
        jnp.dot(hfc, wf2_ref[...], preferred_element_type=jnp.float32)
        + bf2_ref[...]
    )


def _f32(shape):
    return jax.ShapeDtypeStruct(shape, jnp.float32)


_dinv_call = pl.pallas_call(_dinv_tc, out_shape=_f32((1, NP)))
_prescale_call = pl.pallas_call(_prescale_tc, out_shape=_f32((NP, HID)))
_post_call = pl.pallas_call(
    _make_post_tc(False), out_shape=(_f32((NP, HID)), _f32((NP, HID))))
_post_res_call = pl.pallas_call(
    _make_post_tc(True), out_shape=(_f32((NP, HID)), _f32((NP, HID))))
_final_call = pl.pallas_call(_final_tc, out_shape=_f32((G, OUT)))


def kernel(x, edge_index, batch, W1, b1, W2, b2, W3, b3,
           g1, beta1, g2, beta2, g3, beta3, Wf1, bf1, Wf2, bf2):
    src = edge_index[0]
    dst = edge_index[1]
    pad = EP - E
    srcp = jnp.concatenate(
        [src, jnp.full((pad,), N, jnp.int32)]).reshape(EP // CHUNK, CHUNK)
    dstp = jnp.concatenate(
        [dst, jnp.full((pad,), N, jnp.int32)]).reshape(EP // CHUNK, CHUNK)
    xp = jnp.concatenate([x, jnp.zeros((NP - N, D), jnp.float32)], axis=0)
    zero_blk = jnp.zeros((RPT, HID), jnp.float32)

    deg_part = _deg_sc(dstp)
    dinv_col = _dinv_call(deg_part).reshape(NP, 1)

    b1r, b2r, b3r = b1.reshape(1, HID), b2.reshape(1, HID), b3.reshape(1, HID)
    g1r, g2r, g3r = g1.reshape(1, HID), g2.reshape(1, HID), g3.reshape(1, HID)
    be1, be2, be3 = (beta1.reshape(1, HID), beta2.reshape(1, HID),
                     beta3.reshape(1, HID))

    y1 = _prescale_call(xp, W1, dinv_col)
    agg1 = _agg_sc(y1, srcp, dstp, zero_blk)
    h1, y2 = _post_call(agg1, y1, dinv_col, b1r, g1r, be1, W2)
    agg2 = _agg_sc(y2, srcp, dstp, zero_blk)
    h2, y3 = _post_res_call(agg2, y2, dinv_col, b2r, g2r, be2, W3, h1)
    agg3 = _agg_sc(y3, srcp, dstp, zero_blk)
    out = _final_call(agg3, y3, dinv_col, b3r, g3r, be3,
                      batch.reshape(N, 1), Wf1, bf1.reshape(1, HID),
                      Wf2, bf2.reshape(1, OUT))
    return out


# speedup vs baseline: 6.0211x; 6.0211x over previous
import functools

import jax
import jax.numpy as jnp
from jax import lax
from jax.experimental import pallas as pl
from jax.experimental.pallas import tpu as pltpu
from jax.experimental.pallas import tpu_sc as plsc

N = 10000
E = 320000
D = 128
HID = 128
OUT = 2
G = 16
EPS = 1e-5

NC = 2
NS = 16
NW = NC * NS
CHUNK = 128
CHUNKS = 80
DSUP = 16
DOUT = CHUNKS // DSUP
EW = CHUNK * CHUNKS
EP = EW * NW
RPT = 632
NP = RPT * NS

def _agg_sc_body(y_hbm, src_hbm, dst_hbm, zero_hbm, out_hbm,
                 src_v, dstb0, dstb1, rows0, rows1,
                 acc_sh, sem0, sem1, semd0, semd1):
    c = lax.axis_index("c")
    s = lax.axis_index("s")
    wid = c * NS + s
    pltpu.sync_copy(zero_hbm, acc_sh.at[pl.ds(s * RPT, RPT)])
    pltpu.sync_copy(src_hbm.at[pl.ds(wid * CHUNKS, CHUNKS)], src_v)
    plsc.subcore_barrier()

    pltpu.async_copy(dst_hbm.at[pl.ds(wid * CHUNKS, DSUP)], dstb0, semd0)
    pltpu.async_copy(y_hbm.at[src_v.at[0]], rows0, sem0)

    dbufs = ((dstb0, semd0), (dstb1, semd1))
    for o in range(DOUT):
        db, dsem = dbufs[o % 2]
        ndb, ndsem = dbufs[(o + 1) % 2]
        pltpu.make_async_copy(
            dst_hbm.at[pl.ds(wid * CHUNKS + o * DSUP, DSUP)], db, dsem).wait()
        if o + 1 < DOUT:
            pltpu.async_copy(
                dst_hbm.at[pl.ds(wid * CHUNKS + (o + 1) * DSUP, DSUP)],
                ndb, ndsem)

        def body(i, carry):
            for p, buf, sem, obuf, osem in ((0, rows0, sem0, rows1, sem1),
                                            (1, rows1, sem1, rows0, sem0)):
                jj = i * 2 + p
                j = o * DSUP + jj
                nxt = j + 1

                @pl.when(nxt < CHUNKS)
                def _():
                    pltpu.async_copy(y_hbm.at[src_v.at[nxt]], obuf, osem)

                pltpu.make_async_copy(y_hbm.at[src_v.at[j]], buf, sem).wait()
                pltpu.sync_copy(buf, acc_sh.at[db.at[jj]], add=True)
            return carry

        lax.fori_loop(0, DSUP // 2, body, 0)
    plsc.subcore_barrier()
    pltpu.sync_copy(acc_sh.at[pl.ds(s * RPT, RPT)],
                    out_hbm.at[c, pl.ds(s * RPT, RPT)])


def _dinv_tc(degp_ref, o_ref):
    deg = degp_ref[0, :, 0:1] + degp_ref[1, :, 0:1] + 1.0
    rmask = lax.broadcasted_iota(jnp.int32, (NP, 1), 0) < N
    o_ref[...] = jnp.where(rmask, lax.rsqrt(deg), 0.0)


def _prescale_tc(h_ref, w_ref, dinv_ref, o_ref):
    o_ref[...] = (
        jnp.dot(h_ref[...], w_ref[...], preferred_element_type=jnp.float32)
        * dinv_ref[...]
    )


def _bn_relu(z, g_row, beta_row):
    rmask = lax.broadcasted_iota(jnp.int32, (NP, 1), 0) < N
    zm = jnp.where(rmask, z, 0.0)
    mean = jnp.sum(zm, axis=0, keepdims=True) * (1.0 / N)
    cent = z - mean
    var = jnp.sum(jnp.where(rmask, cent * cent, 0.0), axis=0,
                  keepdims=True) * (1.0 / N)
    return jnp.maximum(cent * lax.rsqrt(var + EPS) * g_row + beta_row, 0.0)


def _make_post_tc(with_res):
    def body(*refs):
        if with_res:
            (a_ref, y_ref, dinv_ref, b_ref, g_ref, beta_ref, w_ref,
             res_ref, h_out, y_out) = refs
        else:
            (a_ref, y_ref, dinv_ref, b_ref, g_ref, beta_ref, w_ref,
             h_out, y_out) = refs
        dinv = dinv_ref[...]
        z = dinv * (a_ref[0] + a_ref[1] + y_ref[...]) + b_ref[...]
        h = _bn_relu(z, g_ref[...], beta_ref[...])
        if with_res:
            h = h + res_ref[...]
        h_out[...] = h
        y_out[...] = (
            jnp.dot(h, w_ref[...], preferred_element_type=jnp.float32) * dinv
        )
    return body


def _final_tc(a_ref, y_ref, dinv_ref, b_ref, g_ref, beta_ref, batch_ref,
              wf1_ref, bf1_ref, wf2_ref, bf2_ref, o_ref):
    z = dinv_ref[...] * (a_ref[0] + a_ref[1] + y_ref[...]) + b_ref[...]
    h = _bn_relu(z, g_ref[...], beta_ref[...])
    hn = h[:N]
    bc = batch_ref[...]
    gid = lax.broadcasted_iota(jnp.int32, (1, G), 1)
    onehot = (bc == gid).astype(jnp.float32)
    seg_sum = lax.dot_general(onehot, hn, (((0,), (0,)), ((), ())),
                              preferred_element_type=jnp.float32)
    cnt = lax.dot_general(onehot, jnp.ones((N, 1), jnp.float32),
                          (((0,), (0,)), ((), ())),
                          preferred_element_type=jnp.float32)
    mean_pool = seg_sum / jnp.maximum(cnt, 1.0)
    max_pool = jnp.concatenate(
        [jnp.max(jnp.where(bc == gi, hn, -jnp.inf), axis=0, keepdims=True)
         for gi in range(G)], axis=0)
    pooled = jnp.concatenate([mean_pool, max_pool], axis=1)
    hfc = jnp.maximum(
        jnp.dot(pooled, wf1_ref[...], preferred_element_type=jnp.float32)
        + bf1_ref[...], 0.0)
    o_ref[...] = (
        jnp.dot(hfc, wf2_ref[...], preferred_element_type=jnp.float32)
        + bf2_ref[...]
    )


def _f32(shape):
    return jax.ShapeDtypeStruct(shape, jnp.float32)


@functools.lru_cache(maxsize=1)
def _build_calls():
    mesh = plsc.VectorSubcoreMesh(core_axis_name="c", subcore_axis_name="s")
    agg_sc = pl.kernel(
        _agg_sc_body,
        mesh=mesh,
        out_type=jax.ShapeDtypeStruct((NC, NP, HID), jnp.float32),
        scratch_types=[
            pltpu.VMEM((CHUNKS, CHUNK), jnp.int32),
            pltpu.VMEM((DSUP, CHUNK), jnp.int32),
            pltpu.VMEM((DSUP, CHUNK), jnp.int32),
            pltpu.VMEM((CHUNK, HID), jnp.float32),
            pltpu.VMEM((CHUNK, HID), jnp.float32),
            pltpu.VMEM_SHARED((NP, HID), jnp.float32),
            pltpu.SemaphoreType.DMA,
            pltpu.SemaphoreType.DMA,
            pltpu.SemaphoreType.DMA,
            pltpu.SemaphoreType.DMA,
        ],
    )
    dinv_call = pl.pallas_call(_dinv_tc, out_shape=_f32((NP, 1)))
    prescale_call = pl.pallas_call(_prescale_tc, out_shape=_f32((NP, HID)))
    post_call = pl.pallas_call(
        _make_post_tc(False), out_shape=(_f32((NP, HID)), _f32((NP, HID))))
    post_res_call = pl.pallas_call(
        _make_post_tc(True), out_shape=(_f32((NP, HID)), _f32((NP, HID))))
    final_call = pl.pallas_call(_final_tc, out_shape=_f32((G, OUT)))
    return (agg_sc, dinv_call, prescale_call, post_call,
            post_res_call, final_call)


def kernel(x, edge_index, batch, W1, b1, W2, b2, W3, b3,
           g1, beta1, g2, beta2, g3, beta3, Wf1, bf1, Wf2, bf2):
    (_agg_sc, _dinv_call, _prescale_call, _post_call,
     _post_res_call, _final_call) = _build_calls()
    src = edge_index[0]
    dst = edge_index[1]
    pad = EP - E
    srcp = jnp.concatenate(
        [src, jnp.full((pad,), N, jnp.int32)]).reshape(EP // CHUNK, CHUNK)
    dstp = jnp.concatenate(
        [dst, jnp.full((pad,), N, jnp.int32)]).reshape(EP // CHUNK, CHUNK)
    xp = jnp.concatenate([x, jnp.zeros((NP - N, D), jnp.float32)], axis=0)
    zero_blk = jnp.zeros((RPT, HID), jnp.float32)
    ones_y = jnp.concatenate(
        [jnp.ones((N, HID), jnp.float32),
         jnp.zeros((NP - N, HID), jnp.float32)], axis=0)

    deg_part = _agg_sc(ones_y, srcp, dstp, zero_blk)
    dinv_col = _dinv_call(deg_part)

    b1r, b2r, b3r = b1.reshape(1, HID), b2.reshape(1, HID), b3.reshape(1, HID)
    g1r, g2r, g3r = g1.reshape(1, HID), g2.reshape(1, HID), g3.reshape(1, HID)
    be1, be2, be3 = (beta1.reshape(1, HID), beta2.reshape(1, HID),
                     beta3.reshape(1, HID))

    y1 = _prescale_call(xp, W1, dinv_col)
    agg1 = _agg_sc(y1, srcp, dstp, zero_blk)
    h1, y2 = _post_call(agg1, y1, dinv_col, b1r, g1r, be1, W2)
    agg2 = _agg_sc(y2, srcp, dstp, zero_blk)
    h2, y3 = _post_res_call(agg2, y2, dinv_col, b2r, g2r, be2, W3, h1)
    agg3 = _agg_sc(y3, srcp, dstp, zero_blk)
    out = _final_call(agg3, y3, dinv_col, b3r, g3r, be3,
                      batch.reshape(N, 1), Wf1, bf1.reshape(1, HID),
                      Wf2, bf2.reshape(1, OUT))
    return out


# speedup vs baseline: 7.6298x; 1.2672x over previous
import functools

import jax
import jax.numpy as jnp
from jax import lax
from jax.experimental import pallas as pl
from jax.experimental.pallas import tpu as pltpu
from jax.experimental.pallas import tpu_sc as plsc

N = 10000
E = 320000
D = 128
HID = 128
OUT = 2
G = 16
EPS = 1e-5

NC = 2
NS = 16
NW = NC * NS
CHUNK = 128
CHUNKS = 80
DSUP = 16
DOUT = CHUNKS // DSUP
EW = CHUNK * CHUNKS
EP = EW * NW
RPT = 632
NP = RPT * NS

def _agg_sc_body(y_hbm, src_hbm, dst_hbm, zero_hbm, out_hbm,
                 src_v, dstb0, dstb1, rows0, rows1,
                 acc_sh, sem0, sem1, sems0, sems1, semd0, semd1):
    c = lax.axis_index("c")
    s = lax.axis_index("s")
    wid = c * NS + s
    pltpu.sync_copy(zero_hbm, acc_sh.at[pl.ds(s * RPT, RPT)])
    pltpu.sync_copy(src_hbm.at[pl.ds(wid * CHUNKS, CHUNKS)], src_v)
    plsc.subcore_barrier()

    pltpu.async_copy(dst_hbm.at[pl.ds(wid * CHUNKS, DSUP)], dstb0, semd0)
    pltpu.async_copy(y_hbm.at[src_v.at[0]], rows0, sem0)

    dbufs = ((dstb0, semd0), (dstb1, semd1))
    for o in range(DOUT):
        db, dsem = dbufs[o % 2]
        ndb, ndsem = dbufs[(o + 1) % 2]
        pltpu.make_async_copy(
            dst_hbm.at[pl.ds(wid * CHUNKS + o * DSUP, DSUP)], db, dsem).wait()
        if o + 1 < DOUT:
            pltpu.async_copy(
                dst_hbm.at[pl.ds(wid * CHUNKS + (o + 1) * DSUP, DSUP)],
                ndb, ndsem)

        def body(i, carry):
            for p, buf, gsem, ssem, obuf, ogsem, ossem in (
                    (0, rows0, sem0, sems0, rows1, sem1, sems1),
                    (1, rows1, sem1, sems1, rows0, sem0, sems0)):
                jj = i * 2 + p
                j = o * DSUP + jj
                nxt = j + 1

                @pl.when(nxt < CHUNKS)
                def _():
                    @pl.when(j >= 1)
                    def _():
                        pltpu.make_async_copy(
                            obuf, acc_sh.at[db.at[jj]], ossem).wait()
                    pltpu.async_copy(y_hbm.at[src_v.at[nxt]], obuf, ogsem)

                pltpu.make_async_copy(y_hbm.at[src_v.at[j]], buf, gsem).wait()
                pltpu.async_copy(buf, acc_sh.at[db.at[jj]], ssem, add=True)
            return carry

        lax.fori_loop(0, DSUP // 2, body, 0)
    pltpu.make_async_copy(rows0, acc_sh.at[dstb1.at[0]], sems0).wait()
    pltpu.make_async_copy(rows1, acc_sh.at[dstb1.at[0]], sems1).wait()
    plsc.subcore_barrier()
    pltpu.sync_copy(acc_sh.at[pl.ds(s * RPT, RPT)],
                    out_hbm.at[c, pl.ds(s * RPT, RPT)])


def _deg_sc_body(dst_hbm, ones_hbm, zero_hbm, out_hbm,
                 dst_v, ones_v, acc_sh, sems):
    c = lax.axis_index("c")
    s = lax.axis_index("s")
    wid = c * NS + s
    pltpu.sync_copy(zero_hbm, acc_sh.at[pl.ds(s * RPT, RPT)])
    pltpu.sync_copy(dst_hbm.at[pl.ds(wid * CHUNKS, CHUNKS)], dst_v)
    pltpu.sync_copy(ones_hbm, ones_v)
    plsc.subcore_barrier()

    def body(j, carry):
        pltpu.async_copy(ones_v, acc_sh.at[dst_v.at[j]], sems, add=True)

        @pl.when(j >= 4)
        def _():
            pltpu.make_async_copy(ones_v, acc_sh.at[dst_v.at[j]],
                                  sems).wait()
        return carry

    lax.fori_loop(0, CHUNKS, body, 0)
    for _ in range(4):
        pltpu.make_async_copy(ones_v, acc_sh.at[dst_v.at[0]], sems).wait()
    plsc.subcore_barrier()
    pltpu.sync_copy(acc_sh.at[pl.ds(s * RPT, RPT)],
                    out_hbm.at[c, pl.ds(s * RPT, RPT)])


def _dinv_tc(degp_ref, o_ref):
    deg = degp_ref[0, :, 0:1] + degp_ref[1, :, 0:1] + 1.0
    rmask = lax.broadcasted_iota(jnp.int32, (NP, 1), 0) < N
    o_ref[...] = jnp.where(rmask, lax.rsqrt(deg), 0.0)


def _prescale_tc(h_ref, w_ref, dinv_ref, o_ref):
    o_ref[...] = (
        jnp.dot(h_ref[...], w_ref[...], preferred_element_type=jnp.float32)
        * dinv_ref[...]
    )


def _bn_relu(z, g_row, beta_row):
    rmask = lax.broadcasted_iota(jnp.int32, (NP, 1), 0) < N
    zm = jnp.where(rmask, z, 0.0)
    mean = jnp.sum(zm, axis=0, keepdims=True) * (1.0 / N)
    cent = z - mean
    var = jnp.sum(jnp.where(rmask, cent * cent, 0.0), axis=0,
                  keepdims=True) * (1.0 / N)
    return jnp.maximum(cent * lax.rsqrt(var + EPS) * g_row + beta_row, 0.0)


def _make_post_tc(with_res):
    def body(*refs):
        if with_res:
            (a_ref, y_ref, dinv_ref, b_ref, g_ref, beta_ref, w_ref,
             res_ref, h_out, y_out) = refs
        else:
            (a_ref, y_ref, dinv_ref, b_ref, g_ref, beta_ref, w_ref,
             h_out, y_out) = refs
        dinv = dinv_ref[...]
        z = dinv * (a_ref[0] + a_ref[1] + y_ref[...]) + b_ref[...]
        h = _bn_relu(z, g_ref[...], beta_ref[...])
        if with_res:
            h = h + res_ref[...]
        h_out[...] = h
        y_out[...] = (
            jnp.dot(h, w_ref[...], preferred_element_type=jnp.float32) * dinv
        )
    return body


def _final_tc(a_ref, y_ref, dinv_ref, b_ref, g_ref, beta_ref, batch_ref,
              wf1_ref, bf1_ref, wf2_ref, bf2_ref, o_ref):
    z = dinv_ref[...] * (a_ref[0] + a_ref[1] + y_ref[...]) + b_ref[...]
    h = _bn_relu(z, g_ref[...], beta_ref[...])
    hn = h[:N]
    bc = batch_ref[...]
    gid = lax.broadcasted_iota(jnp.int32, (1, G), 1)
    onehot = (bc == gid).astype(jnp.float32)
    seg_sum = lax.dot_general(onehot, hn, (((0,), (0,)), ((), ())),
                              preferred_element_type=jnp.float32)
    cnt = lax.dot_general(onehot, jnp.ones((N, 1), jnp.float32),
                          (((0,), (0,)), ((), ())),
                          preferred_element_type=jnp.float32)
    mean_pool = seg_sum / jnp.maximum(cnt, 1.0)
    max_pool = jnp.concatenate(
        [jnp.max(jnp.where(bc == gi, hn, -jnp.inf), axis=0, keepdims=True)
         for gi in range(G)], axis=0)
    pooled = jnp.concatenate([mean_pool, max_pool], axis=1)
    hfc = jnp.maximum(
        jnp.dot(pooled, wf1_ref[...], preferred_element_type=jnp.float32)
        + bf1_ref[...], 0.0)
    o_ref[...] = (
        jnp.dot(hfc, wf2_ref[...], preferred_element_type=jnp.float32)
        + bf2_ref[...]
    )


def _f32(shape):
    return jax.ShapeDtypeStruct(shape, jnp.float32)


@functools.lru_cache(maxsize=1)
def _build_calls():
    mesh = plsc.VectorSubcoreMesh(core_axis_name="c", subcore_axis_name="s")
    agg_sc = pl.kernel(
        _agg_sc_body,
        mesh=mesh,
        out_type=jax.ShapeDtypeStruct((NC, NP, HID), jnp.float32),
        scratch_types=[
            pltpu.VMEM((CHUNKS, CHUNK), jnp.int32),
            pltpu.VMEM((DSUP, CHUNK), jnp.int32),
            pltpu.VMEM((DSUP, CHUNK), jnp.int32),
            pltpu.VMEM((CHUNK, HID), jnp.float32),
            pltpu.VMEM((CHUNK, HID), jnp.float32),
            pltpu.VMEM_SHARED((NP, HID), jnp.float32),
            pltpu.SemaphoreType.DMA,
            pltpu.SemaphoreType.DMA,
            pltpu.SemaphoreType.DMA,
            pltpu.SemaphoreType.DMA,
            pltpu.SemaphoreType.DMA,
            pltpu.SemaphoreType.DMA,
        ],
    )
    deg_sc = pl.kernel(
        _deg_sc_body,
        mesh=mesh,
        out_type=jax.ShapeDtypeStruct((NC, NP, HID), jnp.float32),
        scratch_types=[
            pltpu.VMEM((CHUNKS, CHUNK), jnp.int32),
            pltpu.VMEM((CHUNK, HID), jnp.float32),
            pltpu.VMEM_SHARED((NP, HID), jnp.float32),
            pltpu.SemaphoreType.DMA,
        ],
    )
    dinv_call = pl.pallas_call(_dinv_tc, out_shape=_f32((NP, 1)))
    prescale_call = pl.pallas_call(_prescale_tc, out_shape=_f32((NP, HID)))
    post_call = pl.pallas_call(
        _make_post_tc(False), out_shape=(_f32((NP, HID)), _f32((NP, HID))))
    post_res_call = pl.pallas_call(
        _make_post_tc(True), out_shape=(_f32((NP, HID)), _f32((NP, HID))))
    final_call = pl.pallas_call(_final_tc, out_shape=_f32((G, OUT)))
    return (agg_sc, deg_sc, dinv_call, prescale_call, post_call,
            post_res_call, final_call)


def kernel(x, edge_index, batch, W1, b1, W2, b2, W3, b3,
           g1, beta1, g2, beta2, g3, beta3, Wf1, bf1, Wf2, bf2):
    (_agg_sc, _deg_sc, _dinv_call, _prescale_call, _post_call,
     _post_res_call, _final_call) = _build_calls()
    src = edge_index[0]
    dst = edge_index[1]
    pad = EP - E
    srcp = jnp.concatenate(
        [src, jnp.full((pad,), N, jnp.int32)]).reshape(EP // CHUNK, CHUNK)
    dstp = jnp.concatenate(
        [dst, jnp.full((pad,), N, jnp.int32)]).reshape(EP // CHUNK, CHUNK)
    xp = jnp.concatenate([x, jnp.zeros((NP - N, D), jnp.float32)], axis=0)
    zero_blk = jnp.zeros((RPT, HID), jnp.float32)
    ones_blk = jnp.ones((CHUNK, HID), jnp.float32)

    deg_part = _deg_sc(dstp, ones_blk, zero_blk)
    dinv_col = _dinv_call(deg_part)

    b1r, b2r, b3r = b1.reshape(1, HID), b2.reshape(1, HID), b3.reshape(1, HID)
    g1r, g2r, g3r = g1.reshape(1, HID), g2.reshape(1, HID), g3.reshape(1, HID)
    be1, be2, be3 = (beta1.reshape(1, HID), beta2.reshape(1, HID),
                     beta3.reshape(1, HID))

    y1 = _prescale_call(xp, W1, dinv_col)
    agg1 = _agg_sc(y1, srcp, dstp, zero_blk)
    h1, y2 = _post_call(agg1, y1, dinv_col, b1r, g1r, be1, W2)
    agg2 = _agg_sc(y2, srcp, dstp, zero_blk)
    h2, y3 = _post_res_call(agg2, y2, dinv_col, b2r, g2r, be2, W3, h1)
    agg3 = _agg_sc(y3, srcp, dstp, zero_blk)
    out = _final_call(agg3, y3, dinv_col, b3r, g3r, be3,
                      batch.reshape(N, 1), Wf1, bf1.reshape(1, HID),
                      Wf2, bf2.reshape(1, OUT))
    return out
